# baseline (speedup 1.0000x reference)
import jax
import jax.numpy as jnp
from jax.experimental import pallas as pl
from jax.experimental.pallas import tpu as pltpu

_B, _N, _D, _R = 16384, 20, 64, 8
_BS = 256


def _body(u_ref, i_ref, a_ref, o_ref, s_ref, waor_ref, wuir_ref, rp_ref,
          out_ref, c_ref, v_ref):
    @pl.when(pl.program_id(0) == 0)
    def _():
        for r in range(_R):
            wu = wuir_ref[r]
            wa = waor_ref[r]
            c_ref[:, r * 128:(r + 1) * 128] = jax.lax.dot_general(
                wu, wa, (((1,), (1,)), ((), ())),
                preferred_element_type=jnp.float32)
            v_ref[r:r + 1, :] = jnp.sum(
                wa * rp_ref[r:r + 1, :], axis=-1).reshape(1, 2 * _D)

    u = u_ref[:]
    i = i_ref[:]
    g = (jnp.dot(u, c_ref[:_D, :], preferred_element_type=jnp.float32)
         + jnp.dot(i, c_ref[_D:, :], preferred_element_type=jnp.float32))

    s = s_ref[:]
    a = a_ref[:]
    o = o_ref[:]
    pred = None
    for r in range(_R):
        gu = g[:, r * 128:r * 128 + _D] + v_ref[r:r + 1, :_D]
        go = g[:, r * 128 + _D:(r + 1) * 128] + v_ref[r:r + 1, _D:]
        bu = jax.lax.broadcast_in_dim(gu, (_BS, _N, _D), (0, 2))
        bo = jax.lax.broadcast_in_dim(go, (_BS, _N, _D), (0, 2))
        dot_r = jnp.sum(a * bu, axis=-1) + jnp.sum(o * bo, axis=-1)
        pred = dot_r if r == 0 else jnp.where(s == r, dot_r, pred)
    out_ref[:] = pred


def kernel(u_emb, i_emb, a_emb, o_emb, s, w_aor, w_uir, r_param):
    grid = (_B // _BS,)
    return pl.pallas_call(
        _body,
        grid=grid,
        in_specs=[
            pl.BlockSpec((_BS, _D), lambda i: (i, 0)),
            pl.BlockSpec((_BS, _D), lambda i: (i, 0)),
            pl.BlockSpec((_BS, _N, _D), lambda i: (i, 0, 0)),
            pl.BlockSpec((_BS, _N, _D), lambda i: (i, 0, 0)),
            pl.BlockSpec((_BS, _N), lambda i: (i, 0)),
            pl.BlockSpec((_R, 2 * _D, _D), lambda i: (0, 0, 0)),
            pl.BlockSpec((_R, 2 * _D, _D), lambda i: (0, 0, 0)),
            pl.BlockSpec((_R, _D), lambda i: (0, 0)),
        ],
        out_specs=pl.BlockSpec((_BS, _N), lambda i: (i, 0)),
        out_shape=jax.ShapeDtypeStruct((_B, _N), jnp.float32),
        scratch_shapes=[
            pltpu.VMEM((2 * _D, _R * 2 * _D), jnp.float32),
            pltpu.VMEM((_R, 2 * _D), jnp.float32),
        ],
    )(u_emb, i_emb, a_emb, o_emb, s, w_aor, w_uir, r_param)


# speedup vs baseline: 1.3237x; 1.3237x over previous
import jax
import jax.numpy as jnp
from jax.experimental import pallas as pl
from jax.experimental.pallas import tpu as pltpu

_B, _N, _D, _R = 16384, 20, 64, 8
_BS = 256


def _body(u_ref, i_ref, a_ref, o_ref, oh_ref, waor_ref, wuir_ref, rp_ref,
          out_ref, c_ref, v_ref):
    @pl.when(pl.program_id(0) == 0)
    def _():
        for r in range(_R):
            wu = wuir_ref[r]
            wa = waor_ref[r]
            c_ref[:, r * 128:(r + 1) * 128] = jax.lax.dot_general(
                wu, wa, (((1,), (1,)), ((), ())),
                preferred_element_type=jnp.float32)
            v_ref[r:r + 1, :] = jnp.sum(
                wa * rp_ref[r:r + 1, :], axis=-1).reshape(1, 2 * _D)

    u = u_ref[:]
    i = i_ref[:]
    g = (jnp.dot(u, c_ref[:_D, :], preferred_element_type=jnp.float32)
         + jnp.dot(i, c_ref[_D:, :], preferred_element_type=jnp.float32))

    oh = oh_ref[:]
    gsel = None
    for r in range(_R):
        g3r = g[:, r * 128:(r + 1) * 128] + v_ref[r:r + 1, :]
        gb = jax.lax.broadcast_in_dim(g3r, (_BS, _N, 2 * _D), (0, 2))
        ohr = jax.lax.broadcast_in_dim(
            oh[:, :, r:r + 1], (_BS, _N, 2 * _D), (0, 1, 2))
        term = ohr * gb
        gsel = term if r == 0 else gsel + term

    pred = (jnp.sum(a_ref[:] * gsel[:, :, :_D], axis=-1)
            + jnp.sum(o_ref[:] * gsel[:, :, _D:], axis=-1))
    out_ref[:] = pred


def kernel(u_emb, i_emb, a_emb, o_emb, s, w_aor, w_uir, r_param):
    onehot = (s[:, :, None] == jnp.arange(_R, dtype=s.dtype)).astype(
        jnp.float32)
    grid = (_B // _BS,)
    return pl.pallas_call(
        _body,
        grid=grid,
        in_specs=[
            pl.BlockSpec((_BS, _D), lambda i: (i, 0)),
            pl.BlockSpec((_BS, _D), lambda i: (i, 0)),
            pl.BlockSpec((_BS, _N, _D), lambda i: (i, 0, 0)),
            pl.BlockSpec((_BS, _N, _D), lambda i: (i, 0, 0)),
            pl.BlockSpec((_BS, _N, _R), lambda i: (i, 0, 0)),
            pl.BlockSpec((_R, 2 * _D, _D), lambda i: (0, 0, 0)),
            pl.BlockSpec((_R, 2 * _D, _D), lambda i: (0, 0, 0)),
            pl.BlockSpec((_R, _D), lambda i: (0, 0)),
        ],
        out_specs=pl.BlockSpec((_BS, _N), lambda i: (i, 0)),
        out_shape=jax.ShapeDtypeStruct((_B, _N), jnp.float32),
        scratch_shapes=[
            pltpu.VMEM((2 * _D, _R * 2 * _D), jnp.float32),
            pltpu.VMEM((_R, 2 * _D), jnp.float32),
        ],
    )(u_emb, i_emb, a_emb, o_emb, onehot, w_aor, w_uir, r_param)


# speedup vs baseline: 1.8820x; 1.4217x over previous
import jax
import jax.numpy as jnp
from jax.experimental import pallas as pl
from jax.experimental.pallas import tpu as pltpu

_B, _N, _D, _R = 16384, 20, 64, 8
_BS = 256
_TB = _BS * _N
_GB = 16
_NG = _BS // _GB


def _body(u_ref, i_ref, ao_ref, oh_ref, waor_ref, wuir_ref, rp_ref,
          out_ref, c_ref, v_ref):
    @pl.when(pl.program_id(0) == 0)
    def _():
        for r in range(_R):
            wu = wuir_ref[r]
            wa = waor_ref[r]
            c_ref[:, r * 128:(r + 1) * 128] = jax.lax.dot_general(
                wu, wa, (((1,), (1,)), ((), ())),
                preferred_element_type=jnp.float32)
            v_ref[r:r + 1, :] = jnp.sum(
                wa * rp_ref[r:r + 1, :], axis=-1).reshape(1, 2 * _D)

    u = u_ref[:]
    i = i_ref[:]
    g = (jnp.dot(u, c_ref[:_D, :], preferred_element_type=jnp.float32)
         + jnp.dot(i, c_ref[_D:, :], preferred_element_type=jnp.float32))

    parts = []
    for r in range(_R):
        parts.append(
            (g[:, r * 128:(r + 1) * 128] + v_ref[r:r + 1, :])
            .astype(jnp.bfloat16)[None])
    gstack = jnp.concatenate(parts, axis=0)

    gsels = []
    for gi in range(_NG):
        rhs = gstack[:, gi * _GB:(gi + 1) * _GB, :].reshape(_R * _GB, 2 * _D)
        lhs = oh_ref[gi * _GB * _N:(gi + 1) * _GB * _N, :]
        gsels.append(jax.lax.dot_general(
            lhs, rhs, (((1,), (0,)), ((), ())),
            preferred_element_type=jnp.float32))
    gsel = jnp.concatenate(gsels, axis=0)

    out_ref[...] = jnp.sum(ao_ref[:] * gsel, axis=-1)


def kernel(u_emb, i_emb, a_emb, o_emb, s, w_aor, w_uir, r_param):
    ao2 = jnp.concatenate([a_emb, o_emb], axis=-1).reshape(_B * _N, 2 * _D)
    t = jnp.arange(_B * _N, dtype=jnp.int32)
    col = s.reshape(-1) * _GB + (t // _N) % _GB
    ohbd = (col[:, None] == jnp.arange(_R * _GB, dtype=jnp.int32)[None, :]
            ).astype(jnp.bfloat16)

    grid = (_B // _BS,)
    out2 = pl.pallas_call(
        _body,
        grid=grid,
        in_specs=[
            pl.BlockSpec((_BS, _D), lambda i: (i, 0)),
            pl.BlockSpec((_BS, _D), lambda i: (i, 0)),
            pl.BlockSpec((_TB, 2 * _D), lambda i: (i, 0)),
            pl.BlockSpec((_TB, _R * _GB), lambda i: (i, 0)),
            pl.BlockSpec((_R, 2 * _D, _D), lambda i: (0, 0, 0)),
            pl.BlockSpec((_R, 2 * _D, _D), lambda i: (0, 0, 0)),
            pl.BlockSpec((_R, _D), lambda i: (0, 0)),
        ],
        out_specs=pl.BlockSpec((_TB,), lambda i: (i,)),
        out_shape=jax.ShapeDtypeStruct((_B * _N,), jnp.float32),
        scratch_shapes=[
            pltpu.VMEM((2 * _D, _R * 2 * _D), jnp.float32),
            pltpu.VMEM((_R, 2 * _D), jnp.float32),
        ],
    )(u_emb, i_emb, ao2, ohbd, w_aor, w_uir, r_param)
    return out2.reshape(_B, _N)


# speedup vs baseline: 2.0988x; 1.1152x over previous
import jax
import jax.numpy as jnp
from jax.experimental import pallas as pl
from jax.experimental.pallas import tpu as pltpu

_B, _N, _D, _R = 16384, 20, 64, 8
_BS = 256
_TB = _BS * _N
_GB = 16
_NG = _BS // _GB


def _body(u_ref, i_ref, a_ref, o_ref, oh_ref, waor_ref, wuir_ref, rp_ref,
          out_ref, c_ref, v_ref):
    @pl.when(pl.program_id(0) == 0)
    def _():
        for r in range(_R):
            wu = wuir_ref[r]
            wa = waor_ref[r]
            c_ref[:, r * 128:(r + 1) * 128] = jax.lax.dot_general(
                wu, wa, (((1,), (1,)), ((), ())),
                preferred_element_type=jnp.float32)
            v_ref[r:r + 1, :] = jnp.sum(
                wa * rp_ref[r:r + 1, :], axis=-1).reshape(1, 2 * _D)

    u = u_ref[:]
    i = i_ref[:]
    g = (jnp.dot(u, c_ref[:_D, :], preferred_element_type=jnp.float32)
         + jnp.dot(i, c_ref[_D:, :], preferred_element_type=jnp.float32))

    parts = []
    for r in range(_R):
        parts.append(
            (g[:, r * 128:(r + 1) * 128] + v_ref[r:r + 1, :])
            .astype(jnp.bfloat16)[None])
    gstack = jnp.concatenate(parts, axis=0)

    gsels = []
    for gi in range(_NG):
        rhs = gstack[:, gi * _GB:(gi + 1) * _GB, :].reshape(_R * _GB, 2 * _D)
        lhs = oh_ref[gi * _GB * _N:(gi + 1) * _GB * _N, :]
        gsels.append(jax.lax.dot_general(
            lhs, rhs, (((1,), (0,)), ((), ())),
            preferred_element_type=jnp.float32))
    gsel = jnp.concatenate(gsels, axis=0)

    ao2 = jnp.concatenate([a_ref[:], o_ref[:]], axis=-1).reshape(_TB, 2 * _D)
    prod = ao2 * gsel

    ones = jnp.ones((1, 2 * _D), jnp.float32)
    preds = []
    for c in range(_TB // 128):
        preds.append(jax.lax.dot_general(
            ones, prod[c * 128:(c + 1) * 128, :],
            (((1,), (1,)), ((), ())), preferred_element_type=jnp.float32))
    out_ref[...] = jnp.concatenate(preds, axis=0)


def kernel(u_emb, i_emb, a_emb, o_emb, s, w_aor, w_uir, r_param):
    t = jnp.arange(_B * _N, dtype=jnp.int32)
    col = s.reshape(-1) * _GB + (t // _N) % _GB
    ohbd = (col[:, None] == jnp.arange(_R * _GB, dtype=jnp.int32)[None, :]
            ).astype(jnp.bfloat16)

    grid = (_B // _BS,)
    out2 = pl.pallas_call(
        _body,
        grid=grid,
        in_specs=[
            pl.BlockSpec((_BS, _D), lambda i: (i, 0)),
            pl.BlockSpec((_BS, _D), lambda i: (i, 0)),
            pl.BlockSpec((_BS, _N, _D), lambda i: (i, 0, 0)),
            pl.BlockSpec((_BS, _N, _D), lambda i: (i, 0, 0)),
            pl.BlockSpec((_TB, _R * _GB), lambda i: (i, 0)),
            pl.BlockSpec((_R, 2 * _D, _D), lambda i: (0, 0, 0)),
            pl.BlockSpec((_R, 2 * _D, _D), lambda i: (0, 0, 0)),
            pl.BlockSpec((_R, _D), lambda i: (0, 0)),
        ],
        out_specs=pl.BlockSpec((_TB // 128, 128), lambda i: (i, 0)),
        out_shape=jax.ShapeDtypeStruct((_B * _N // 128, 128), jnp.float32),
        scratch_shapes=[
            pltpu.VMEM((2 * _D, _R * 2 * _D), jnp.float32),
            pltpu.VMEM((_R, 2 * _D), jnp.float32),
        ],
    )(u_emb, i_emb, a_emb, o_emb, ohbd, w_aor, w_uir, r_param)
    return out2.reshape(_B, _N)


# speedup vs baseline: 2.8999x; 1.3817x over previous
import jax
import jax.numpy as jnp
from jax.experimental import pallas as pl
from jax.experimental.pallas import tpu as pltpu

_B, _N, _D, _R = 16384, 20, 64, 8
_BS = 512
_TB = _BS * _N
_GB = 16
_NG = _BS // _GB


def _body(u_ref, i_ref, a_ref, o_ref, s_ref, waor_ref, wuir_ref, rp_ref,
          out_ref, c_ref, v_ref):
    @pl.when(pl.program_id(0) == 0)
    def _():
        for r in range(_R):
            wu = wuir_ref[r]
            wa = waor_ref[r]
            c_ref[:, r * 128:(r + 1) * 128] = jax.lax.dot_general(
                wu, wa, (((1,), (1,)), ((), ())),
                preferred_element_type=jnp.float32)
            v_ref[r:r + 1, :] = jnp.sum(
                wa * rp_ref[r:r + 1, :], axis=-1).reshape(1, 2 * _D)

    u = u_ref[:]
    i = i_ref[:]
    g = (jnp.dot(u, c_ref[:_D, :], preferred_element_type=jnp.float32)
         + jnp.dot(i, c_ref[_D:, :], preferred_element_type=jnp.float32))

    parts = []
    for r in range(_R):
        parts.append(
            (g[:, r * 128:(r + 1) * 128] + v_ref[r:r + 1, :])
            .astype(jnp.bfloat16)[None])
    gstack = jnp.concatenate(parts, axis=0)

    s3 = jax.lax.broadcast_in_dim(s_ref[:], (_BS, _N, _R * _GB), (0, 1))
    bmod = jax.lax.broadcasted_iota(jnp.int32, (_BS, _N, _R * _GB), 0) % _GB
    lane = jax.lax.broadcasted_iota(jnp.int32, (_BS, _N, _R * _GB), 2)
    oh3 = (lane == s3 * _GB + bmod).astype(jnp.float32)
    ohbd = oh3.reshape(_TB, _R * _GB).astype(jnp.bfloat16)

    gsels = []
    for gi in range(_NG):
        rhs = gstack[:, gi * _GB:(gi + 1) * _GB, :].reshape(_R * _GB, 2 * _D)
        lhs = ohbd[gi * _GB * _N:(gi + 1) * _GB * _N, :]
        gsels.append(jax.lax.dot_general(
            lhs, rhs, (((1,), (0,)), ((), ())),
            preferred_element_type=jnp.float32))
    gsel = jnp.concatenate(gsels, axis=0)

    ao2 = jnp.concatenate([a_ref[:], o_ref[:]], axis=-1).reshape(_TB, 2 * _D)
    prod = ao2 * gsel

    ones = jnp.ones((1, 2 * _D), jnp.float32)
    preds = []
    for c in range(_TB // 128):
        preds.append(jax.lax.dot_general(
            ones, prod[c * 128:(c + 1) * 128, :],
            (((1,), (1,)), ((), ())), preferred_element_type=jnp.float32))
    out_ref[...] = jnp.concatenate(preds, axis=0)


def kernel(u_emb, i_emb, a_emb, o_emb, s, w_aor, w_uir, r_param):
    grid = (_B // _BS,)
    out2 = pl.pallas_call(
        _body,
        grid=grid,
        in_specs=[
            pl.BlockSpec((_BS, _D), lambda i: (i, 0)),
            pl.BlockSpec((_BS, _D), lambda i: (i, 0)),
            pl.BlockSpec((_BS, _N, _D), lambda i: (i, 0, 0)),
            pl.BlockSpec((_BS, _N, _D), lambda i: (i, 0, 0)),
            pl.BlockSpec((_BS, _N), lambda i: (i, 0)),
            pl.BlockSpec((_R, 2 * _D, _D), lambda i: (0, 0, 0)),
            pl.BlockSpec((_R, 2 * _D, _D), lambda i: (0, 0, 0)),
            pl.BlockSpec((_R, _D), lambda i: (0, 0)),
        ],
        out_specs=pl.BlockSpec((_TB // 128, 128), lambda i: (i, 0)),
        out_shape=jax.ShapeDtypeStruct((_B * _N // 128, 128), jnp.float32),
        scratch_shapes=[
            pltpu.VMEM((2 * _D, _R * 2 * _D), jnp.float32),
            pltpu.VMEM((_R, 2 * _D), jnp.float32),
        ],
    )(u_emb, i_emb, a_emb, o_emb, s, w_aor, w_uir, r_param)
    return out2.reshape(_B, _N)
